# baseline (speedup 1.0000x reference)
import functools

import jax
import jax.numpy as jnp
from jax import lax
from jax.experimental import pallas as pl
from jax.experimental.pallas import tpu as pltpu
from jax.experimental.pallas import tpu_sc as plsc

_N = 10000
_D_IN = 128
_D_H = 64
_ROW_BLK = 2000
_SIM_BLK = 400

_NC = 2
_NS = 16
_NW = _NC * _NS
_CHUNK = 128
_E = 320000
_NBUF = 5
_CH = 80
_E_PAD = _NW * _CH * _CHUNK
_NROWS = 10112
_ZROWS = _NROWS // _NS
_DUMMY = 10008


def _leaky_relu(v):
    return jnp.where(v >= 0, v, 0.01 * v)


def _proj_body(x_ref, wlt_ref, wrt_ref, b_ref, p_ref, r_ref):
    xb = x_ref[...]
    p_ref[...] = jnp.dot(xb, wlt_ref[...], preferred_element_type=jnp.float32)
    r_ref[...] = jnp.dot(xb, wrt_ref[...], preferred_element_type=jnp.float32) + b_ref[...]


def _proj(x, wlt, wrt, b):
    n, d = x.shape
    dh = wlt.shape[1]
    grid = n // _ROW_BLK
    return pl.pallas_call(
        _proj_body,
        grid=(grid,),
        in_specs=[
            pl.BlockSpec((_ROW_BLK, d), lambda i: (i, 0)),
            pl.BlockSpec((d, dh), lambda i: (0, 0)),
            pl.BlockSpec((d, dh), lambda i: (0, 0)),
            pl.BlockSpec((1, dh), lambda i: (0, 0)),
        ],
        out_specs=[
            pl.BlockSpec((_ROW_BLK, dh), lambda i: (i, 0)),
            pl.BlockSpec((_ROW_BLK, dh), lambda i: (i, 0)),
        ],
        out_shape=[
            jax.ShapeDtypeStruct((n, dh), jnp.float32),
            jax.ShapeDtypeStruct((n, dh), jnp.float32),
        ],
    )(x, wlt, wrt, b)


def _combine_proj_body(agg_ref, cnt_ref, r_ref,
                       wlt_ref, wrt_ref, b_ref, p_ref, rr_ref):
    cnt = cnt_ref[0, :, 0:1] + cnt_ref[1, :, 0:1]
    inv = 1.0 / jnp.maximum(cnt, 1.0)
    h = _leaky_relu((agg_ref[0] + agg_ref[1]) * inv + r_ref[...])
    p_ref[...] = jnp.dot(h, wlt_ref[...], preferred_element_type=jnp.float32)
    rr_ref[...] = jnp.dot(h, wrt_ref[...], preferred_element_type=jnp.float32) + b_ref[...]


def _combine_proj(agg, cnt, r, wlt, wrt, b):
    n, dh = r.shape
    grid = n // _ROW_BLK
    return pl.pallas_call(
        _combine_proj_body,
        grid=(grid,),
        in_specs=[
            pl.BlockSpec((_NC, _ROW_BLK, dh), lambda i: (0, i, 0)),
            pl.BlockSpec((_NC, _ROW_BLK, 16), lambda i: (0, i, 0)),
            pl.BlockSpec((_ROW_BLK, dh), lambda i: (i, 0)),
            pl.BlockSpec((dh, dh), lambda i: (0, 0)),
            pl.BlockSpec((dh, dh), lambda i: (0, 0)),
            pl.BlockSpec((1, dh), lambda i: (0, 0)),
        ],
        out_specs=[
            pl.BlockSpec((_ROW_BLK, dh), lambda i: (i, 0)),
            pl.BlockSpec((_ROW_BLK, dh), lambda i: (i, 0)),
        ],
        out_shape=[
            jax.ShapeDtypeStruct((n, dh), jnp.float32),
            jax.ShapeDtypeStruct((n, dh), jnp.float32),
        ],
    )(agg, cnt, r, wlt, wrt, b)


def _combine_fc_body(agg_ref, cnt_ref, r_ref, wt_ref, b_ref, h_ref):
    cnt = cnt_ref[0, :, 0:1] + cnt_ref[1, :, 0:1]
    inv = 1.0 / jnp.maximum(cnt, 1.0)
    g = _leaky_relu((agg_ref[0] + agg_ref[1]) * inv + r_ref[...])
    h_ref[...] = jnp.dot(g, wt_ref[...], preferred_element_type=jnp.float32) + b_ref[...]


def _combine_fc(agg, cnt, r, wt, b):
    n, dh = r.shape
    grid = n // _ROW_BLK
    return pl.pallas_call(
        _combine_fc_body,
        grid=(grid,),
        in_specs=[
            pl.BlockSpec((_NC, _ROW_BLK, dh), lambda i: (0, i, 0)),
            pl.BlockSpec((_NC, _ROW_BLK, 16), lambda i: (0, i, 0)),
            pl.BlockSpec((_ROW_BLK, dh), lambda i: (i, 0)),
            pl.BlockSpec((dh, dh), lambda i: (0, 0)),
            pl.BlockSpec((1, dh), lambda i: (0, 0)),
        ],
        out_specs=pl.BlockSpec((_ROW_BLK, dh), lambda i: (i, 0)),
        out_shape=jax.ShapeDtypeStruct((n, dh), jnp.float32),
    )(agg, cnt, r, wt, b)


def _sim_body(ha_ref, hb_ref, out_ref):
    s = jax.lax.dot_general(ha_ref[...], hb_ref[...],
                            (((1,), (1,)), ((), ())),
                            preferred_element_type=jnp.float32)
    out_ref[...] = jax.nn.sigmoid(s)


def _sim(h):
    n, dh = h.shape
    grid = n // _SIM_BLK
    return pl.pallas_call(
        _sim_body,
        grid=(grid,),
        in_specs=[
            pl.BlockSpec((_SIM_BLK, dh), lambda i: (i, 0)),
            pl.BlockSpec((n, dh), lambda i: (0, 0)),
        ],
        out_specs=pl.BlockSpec((_SIM_BLK, n), lambda i: (i, 0)),
        out_shape=jax.ShapeDtypeStruct((n, n), jnp.float32),
    )(h, h)


_SC_MESH = plsc.VectorSubcoreMesh(
    core_axis_name="c", subcore_axis_name="s", num_cores=_NC, num_subcores=_NS)


def _sc_agg(p, src_g, dst_g, zeros_agg, zeros_cnt, ones, with_cnt):
    dh = p.shape[1]
    out_type = [jax.ShapeDtypeStruct((_NC, _NROWS, dh), jnp.float32)]
    scratch = [
        pltpu.VMEM((_CH, _CHUNK), jnp.int32),
        pltpu.VMEM((_CH, _CHUNK), jnp.int32),
        pltpu.VMEM((_NBUF, _CHUNK, dh), jnp.float32),
        pltpu.VMEM_SHARED((_NROWS, dh), jnp.float32),
    ] + [pltpu.SemaphoreType.DMA] * _NBUF
    if with_cnt:
        out_type.append(jax.ShapeDtypeStruct((_NC, _NROWS, 16), jnp.float32))
        scratch += [
            pltpu.VMEM((_CHUNK, 16), jnp.float32),
            pltpu.VMEM_SHARED((_NROWS, 16), jnp.float32),
        ]

    @functools.partial(
        pl.kernel, out_type=out_type, mesh=_SC_MESH, scratch_types=scratch,
        compiler_params=pltpu.CompilerParams(use_tc_tiling_on_sc=False))
    def k(*refs):
        if with_cnt:
            (p_hbm, src_hbm, dst_hbm, za_hbm, zc_hbm, ones_hbm,
             agg_out, cnt_out, src_v, dst_v, rows_v, agg_sh,
             g0, g1, g2, g3, g4, ones_v, cnt_sh) = refs
        else:
            (p_hbm, src_hbm, dst_hbm, za_hbm, agg_out, src_v, dst_v, rows_v,
             agg_sh, g0, g1, g2, g3, g4) = refs
        gsems = (g0, g1, g2, g3, g4)
        c = lax.axis_index("c")
        s = lax.axis_index("s")
        wid = c * _NS + s
        pltpu.sync_copy(za_hbm, agg_sh.at[pl.ds(s * _ZROWS, _ZROWS), :])
        if with_cnt:
            pltpu.sync_copy(zc_hbm, cnt_sh.at[pl.ds(s * _ZROWS, _ZROWS), :])
            pltpu.sync_copy(ones_hbm, ones_v)
        pltpu.sync_copy(src_hbm.at[wid], src_v)
        pltpu.sync_copy(dst_hbm.at[wid], dst_v)

        for b in range(_NBUF):
            pltpu.async_copy(p_hbm.at[src_v.at[b]], rows_v.at[b], gsems[b])
        plsc.subcore_barrier()

        @pl.loop(0, _CH, step=_NBUF)
        def _(g):
            for b in range(_NBUF):
                j = g + b
                pltpu.make_async_copy(
                    p_hbm.at[src_v.at[j]], rows_v.at[b], gsems[b]).wait()
                pltpu.sync_copy(rows_v.at[b], agg_sh.at[dst_v.at[j]], add=True)
                if with_cnt:
                    pltpu.sync_copy(ones_v, cnt_sh.at[dst_v.at[j]], add=True)

                @pl.when(j + _NBUF < _CH)
                def _():
                    pltpu.async_copy(p_hbm.at[src_v.at[j + _NBUF]],
                                     rows_v.at[b], gsems[b])

        plsc.subcore_barrier()
        pltpu.sync_copy(agg_sh.at[pl.ds(s * _ZROWS, _ZROWS), :],
                        agg_out.at[c, pl.ds(s * _ZROWS, _ZROWS), :])
        if with_cnt:
            pltpu.sync_copy(cnt_sh.at[pl.ds(s * _ZROWS, _ZROWS), :],
                            cnt_out.at[c, pl.ds(s * _ZROWS, _ZROWS), :])

    if with_cnt:
        return k(p, src_g, dst_g, zeros_agg, zeros_cnt, ones)
    return k(p, src_g, dst_g, zeros_agg)


def kernel(x, edge_index, W1l, b1, W1r, W2l, b2, W2r, Wfc, bfc):
    pad = _E_PAD - _E
    src_g = jnp.concatenate(
        [edge_index[0], jnp.zeros((pad,), jnp.int32)]).reshape(_NW, _CH, _CHUNK)
    dst_g = jnp.concatenate(
        [edge_index[1], jnp.full((pad,), _DUMMY, jnp.int32)]).reshape(_NW, _CH, _CHUNK)
    zeros_agg = jnp.zeros((_ZROWS, _D_H), jnp.float32)
    zeros_cnt = jnp.zeros((_ZROWS, 16), jnp.float32)
    ones = jnp.ones((_CHUNK, 16), jnp.float32)

    p1, r1 = _proj(x, W1l.T, W1r.T, b1[None, :])
    agg1, cnt_p = _sc_agg(p1, src_g, dst_g, zeros_agg, zeros_cnt, ones, True)

    p2, r2 = _combine_proj(agg1, cnt_p, r1, W2l.T, W2r.T, b2[None, :])
    (agg2,) = _sc_agg(p2, src_g, dst_g, zeros_agg, None, None, False)

    h = _combine_fc(agg2, cnt_p, r2, Wfc.T, bfc[None, :])
    return _sim(h)


# speedup vs baseline: 1.7793x; 1.7793x over previous
import functools

import jax
import jax.numpy as jnp
from jax import lax
from jax.experimental import pallas as pl
from jax.experimental.pallas import tpu as pltpu
from jax.experimental.pallas import tpu_sc as plsc

_N = 10000
_D_IN = 128
_D_H = 64
_ROW_BLK = 2000
_SIM_BLK = 400

_NC = 2
_NS = 16
_NW = _NC * _NS
_CHUNK = 128
_E = 320000
_NBUF = 2
_CH = 80
_E_PAD = _NW * _CH * _CHUNK
_NROWS = 10112
_ZROWS = _NROWS // _NS
_DUMMY = 10008


def _leaky_relu(v):
    return jnp.where(v >= 0, v, 0.01 * v)


def _proj_body(x_ref, wlt_ref, wrt_ref, b_ref, p_ref, r_ref):
    xb = x_ref[...]
    p_ref[...] = jnp.dot(xb, wlt_ref[...], preferred_element_type=jnp.float32)
    r_ref[...] = jnp.dot(xb, wrt_ref[...], preferred_element_type=jnp.float32) + b_ref[...]


def _proj(x, wlt, wrt, b):
    n, d = x.shape
    dh = wlt.shape[1]
    grid = n // _ROW_BLK
    return pl.pallas_call(
        _proj_body,
        grid=(grid,),
        in_specs=[
            pl.BlockSpec((_ROW_BLK, d), lambda i: (i, 0)),
            pl.BlockSpec((d, dh), lambda i: (0, 0)),
            pl.BlockSpec((d, dh), lambda i: (0, 0)),
            pl.BlockSpec((1, dh), lambda i: (0, 0)),
        ],
        out_specs=[
            pl.BlockSpec((_ROW_BLK, dh), lambda i: (i, 0)),
            pl.BlockSpec((_ROW_BLK, dh), lambda i: (i, 0)),
        ],
        out_shape=[
            jax.ShapeDtypeStruct((n, dh), jnp.float32),
            jax.ShapeDtypeStruct((n, dh), jnp.float32),
        ],
    )(x, wlt, wrt, b)


def _combine_proj_body(agg_ref, cnt_ref, r_ref,
                       wlt_ref, wrt_ref, b_ref, p_ref, rr_ref):
    cnt = cnt_ref[0, :, 0:1] + cnt_ref[1, :, 0:1]
    inv = 1.0 / jnp.maximum(cnt, 1.0)
    h = _leaky_relu((agg_ref[0] + agg_ref[1]) * inv + r_ref[...])
    p_ref[...] = jnp.dot(h, wlt_ref[...], preferred_element_type=jnp.float32)
    rr_ref[...] = jnp.dot(h, wrt_ref[...], preferred_element_type=jnp.float32) + b_ref[...]


def _combine_proj(agg, cnt, r, wlt, wrt, b):
    n, dh = r.shape
    grid = n // _ROW_BLK
    return pl.pallas_call(
        _combine_proj_body,
        grid=(grid,),
        in_specs=[
            pl.BlockSpec((_NC, _ROW_BLK, dh), lambda i: (0, i, 0)),
            pl.BlockSpec((_NC, _ROW_BLK, 16), lambda i: (0, i, 0)),
            pl.BlockSpec((_ROW_BLK, dh), lambda i: (i, 0)),
            pl.BlockSpec((dh, dh), lambda i: (0, 0)),
            pl.BlockSpec((dh, dh), lambda i: (0, 0)),
            pl.BlockSpec((1, dh), lambda i: (0, 0)),
        ],
        out_specs=[
            pl.BlockSpec((_ROW_BLK, dh), lambda i: (i, 0)),
            pl.BlockSpec((_ROW_BLK, dh), lambda i: (i, 0)),
        ],
        out_shape=[
            jax.ShapeDtypeStruct((n, dh), jnp.float32),
            jax.ShapeDtypeStruct((n, dh), jnp.float32),
        ],
    )(agg, cnt, r, wlt, wrt, b)


def _combine_fc_body(agg_ref, cnt_ref, r_ref, wt_ref, b_ref, h_ref):
    cnt = cnt_ref[0, :, 0:1] + cnt_ref[1, :, 0:1]
    inv = 1.0 / jnp.maximum(cnt, 1.0)
    g = _leaky_relu((agg_ref[0] + agg_ref[1]) * inv + r_ref[...])
    h_ref[...] = jnp.dot(g, wt_ref[...], preferred_element_type=jnp.float32) + b_ref[...]


def _combine_fc(agg, cnt, r, wt, b):
    n, dh = r.shape
    grid = n // _ROW_BLK
    return pl.pallas_call(
        _combine_fc_body,
        grid=(grid,),
        in_specs=[
            pl.BlockSpec((_NC, _ROW_BLK, dh), lambda i: (0, i, 0)),
            pl.BlockSpec((_NC, _ROW_BLK, 16), lambda i: (0, i, 0)),
            pl.BlockSpec((_ROW_BLK, dh), lambda i: (i, 0)),
            pl.BlockSpec((dh, dh), lambda i: (0, 0)),
            pl.BlockSpec((1, dh), lambda i: (0, 0)),
        ],
        out_specs=pl.BlockSpec((_ROW_BLK, dh), lambda i: (i, 0)),
        out_shape=jax.ShapeDtypeStruct((n, dh), jnp.float32),
    )(agg, cnt, r, wt, b)


def _sim_body(ha_ref, hb_ref, out_ref):
    s = jax.lax.dot_general(ha_ref[...], hb_ref[...],
                            (((1,), (1,)), ((), ())),
                            preferred_element_type=jnp.float32)
    out_ref[...] = jax.nn.sigmoid(s)


def _sim(h):
    n, dh = h.shape
    grid = n // _SIM_BLK
    return pl.pallas_call(
        _sim_body,
        grid=(grid,),
        in_specs=[
            pl.BlockSpec((_SIM_BLK, dh), lambda i: (i, 0)),
            pl.BlockSpec((n, dh), lambda i: (0, 0)),
        ],
        out_specs=pl.BlockSpec((_SIM_BLK, n), lambda i: (i, 0)),
        out_shape=jax.ShapeDtypeStruct((n, n), jnp.float32),
    )(h, h)


_SC_MESH = plsc.VectorSubcoreMesh(
    core_axis_name="c", subcore_axis_name="s", num_cores=_NC, num_subcores=_NS)


def _sc_agg(p, src_g, dst_g, zeros_agg, zeros_cnt, ones, with_cnt):
    dh = p.shape[1]
    n = p.shape[0]
    prows = n // _NS
    out_type = [jax.ShapeDtypeStruct((_NC, _NROWS, dh), jnp.float32)]
    scratch = [
        pltpu.VMEM((_CH, _CHUNK), jnp.int32),
        pltpu.VMEM((_CH, _CHUNK), jnp.int32),
        pltpu.VMEM((_NBUF, _CHUNK, dh), jnp.float32),
        pltpu.VMEM_SHARED((n, dh), jnp.float32),
        pltpu.VMEM_SHARED((_NROWS, dh), jnp.float32),
    ] + [pltpu.SemaphoreType.DMA] * _NBUF
    if with_cnt:
        out_type.append(jax.ShapeDtypeStruct((_NC, _NROWS, 16), jnp.float32))
        scratch += [
            pltpu.VMEM((_CHUNK, 16), jnp.float32),
            pltpu.VMEM_SHARED((_NROWS, 16), jnp.float32),
        ]

    @functools.partial(
        pl.kernel, out_type=out_type, mesh=_SC_MESH, scratch_types=scratch,
        compiler_params=pltpu.CompilerParams(use_tc_tiling_on_sc=False))
    def k(*refs):
        if with_cnt:
            (p_hbm, src_hbm, dst_hbm, za_hbm, zc_hbm, ones_hbm,
             agg_out, cnt_out, src_v, dst_v, rows_v, p_sh, agg_sh,
             g0, g1, ones_v, cnt_sh) = refs
        else:
            (p_hbm, src_hbm, dst_hbm, za_hbm, agg_out, src_v, dst_v, rows_v,
             p_sh, agg_sh, g0, g1) = refs
        gsems = (g0, g1)
        c = lax.axis_index("c")
        s = lax.axis_index("s")
        wid = c * _NS + s
        pltpu.sync_copy(p_hbm.at[pl.ds(s * prows, prows), :],
                        p_sh.at[pl.ds(s * prows, prows), :])
        pltpu.sync_copy(za_hbm, agg_sh.at[pl.ds(s * _ZROWS, _ZROWS), :])
        if with_cnt:
            pltpu.sync_copy(zc_hbm, cnt_sh.at[pl.ds(s * _ZROWS, _ZROWS), :])
            pltpu.sync_copy(ones_hbm, ones_v)
        pltpu.sync_copy(src_hbm.at[wid], src_v)
        pltpu.sync_copy(dst_hbm.at[wid], dst_v)

        plsc.subcore_barrier()
        for b in range(_NBUF):
            pltpu.async_copy(p_sh.at[src_v.at[b]], rows_v.at[b], gsems[b])

        @pl.loop(0, _CH, step=_NBUF)
        def _(g):
            for b in range(_NBUF):
                j = g + b
                pltpu.make_async_copy(
                    p_sh.at[src_v.at[j]], rows_v.at[b], gsems[b]).wait()
                pltpu.sync_copy(rows_v.at[b], agg_sh.at[dst_v.at[j]], add=True)
                if with_cnt:
                    pltpu.sync_copy(ones_v, cnt_sh.at[dst_v.at[j]], add=True)

                @pl.when(j + _NBUF < _CH)
                def _():
                    pltpu.async_copy(p_sh.at[src_v.at[j + _NBUF]],
                                     rows_v.at[b], gsems[b])

        plsc.subcore_barrier()
        pltpu.sync_copy(agg_sh.at[pl.ds(s * _ZROWS, _ZROWS), :],
                        agg_out.at[c, pl.ds(s * _ZROWS, _ZROWS), :])
        if with_cnt:
            pltpu.sync_copy(cnt_sh.at[pl.ds(s * _ZROWS, _ZROWS), :],
                            cnt_out.at[c, pl.ds(s * _ZROWS, _ZROWS), :])

    if with_cnt:
        return k(p, src_g, dst_g, zeros_agg, zeros_cnt, ones)
    return k(p, src_g, dst_g, zeros_agg)


def kernel(x, edge_index, W1l, b1, W1r, W2l, b2, W2r, Wfc, bfc):
    pad = _E_PAD - _E
    src_g = jnp.concatenate(
        [edge_index[0], jnp.zeros((pad,), jnp.int32)]).reshape(_NW, _CH, _CHUNK)
    dst_g = jnp.concatenate(
        [edge_index[1], jnp.full((pad,), _DUMMY, jnp.int32)]).reshape(_NW, _CH, _CHUNK)
    zeros_agg = jnp.zeros((_ZROWS, _D_H), jnp.float32)
    zeros_cnt = jnp.zeros((_ZROWS, 16), jnp.float32)
    ones = jnp.ones((_CHUNK, 16), jnp.float32)

    p1, r1 = _proj(x, W1l.T, W1r.T, b1[None, :])
    agg1, cnt_p = _sc_agg(p1, src_g, dst_g, zeros_agg, zeros_cnt, ones, True)

    p2, r2 = _combine_proj(agg1, cnt_p, r1, W2l.T, W2r.T, b2[None, :])
    (agg2,) = _sc_agg(p2, src_g, dst_g, zeros_agg, None, None, False)

    h = _combine_fc(agg2, cnt_p, r2, Wfc.T, bfc[None, :])
    return _sim(h)
